# baseline (speedup 1.0000x reference)
import jax
import jax.numpy as jnp
from jax.experimental import pallas as pl
from jax.experimental.pallas import tpu as pltpu

BT = 256
PH = 384
NCOL = 2 * PH - 124


def _fused_kernel(xd_ref, ws_ref, bc_ref, w1_ref, b1_ref, w2_ref, b2_ref,
                  o_ref, pl0_ref, pl1_ref):
    bt = o_ref.shape[0]
    pl0_ref[...] = jnp.zeros(pl0_ref.shape, jnp.float32)
    pl1_ref[...] = jnp.zeros(pl1_ref.shape, jnp.float32)
    pl0_ref[1:15, :, 1:29] = xd_ref[1]
    pl1_ref[0:14, :, 1:29] = xd_ref[0]

    s_all = jnp.concatenate(
        [jnp.concatenate(
            [pl0_ref[0:13], pl1_ref[0:13], pl0_ref[1:14],
             pl1_ref[1:14], pl0_ref[2:15]], axis=2),
         jnp.concatenate(
            [pl1_ref[0:13], pl0_ref[1:14], pl1_ref[1:14],
             pl0_ref[2:15], pl1_ref[2:15]], axis=2)], axis=0)

    cc = jnp.dot(s_all.reshape(26 * bt, 150), ws_ref[...],
                 preferred_element_type=jnp.float32)

    t = jnp.maximum(cc[:13 * bt], cc[13 * bt:])
    m = jnp.maximum(t[:, 0:260], t[:, PH:PH + 260])
    m = jnp.maximum(m + bc_ref[...], 0.0)

    mi = m.reshape(13, bt, 260)
    h = b1_ref[...]
    for i in range(13):
        h = h + jnp.dot(mi[i], w1_ref[i],
                        preferred_element_type=jnp.float32)
    h = jnp.maximum(h, 0.0)
    o_ref[...] = (jnp.dot(h, w2_ref[...], preferred_element_type=jnp.float32)
                  + b2_ref[...])


@jax.jit
def _forward(x, conv_w, conv_b, fc1_w, fc1_b, fc2_w, fc2_b):
    B = x.shape[0]
    x3 = x.reshape(B, 28, 28)
    Bp = -(-B // BT) * BT
    if Bp != B:
        x3 = jnp.pad(x3, ((0, Bp - B), (0, 0), (0, 0)))
    xd = x3.reshape(Bp, 14, 2, 28).transpose(2, 1, 0, 3)

    wc = conv_w.reshape(20, 5, 5)
    c = jnp.arange(20)[:, None, None, None, None]
    j = jnp.arange(13)[None, :, None, None, None]
    ph = jnp.arange(2)[None, None, :, None, None]
    ky = jnp.arange(5)[None, None, None, :, None]
    kx = jnp.arange(5)[None, None, None, None, :]
    full = (20, 13, 2, 5, 5)
    rows = jnp.broadcast_to(ky * 30 + 2 * j + ph + kx, full)
    cols = jnp.broadcast_to(ph * PH + c * 13 + j, full)
    vals = jnp.broadcast_to(wc[c, ky, kx], full)
    ws = jnp.zeros((150, NCOL), jnp.float32).at[
        rows.reshape(-1), cols.reshape(-1)].set(vals.reshape(-1))

    bc = jnp.broadcast_to(conv_b[:, None], (20, 13)).reshape(1, 260)

    w1r = fc1_w.reshape(100, 20, 13, 13).transpose(2, 1, 3, 0)
    w1r = w1r.reshape(13, 260, 100)
    b1 = fc1_b.reshape(1, 100)
    w2 = fc2_w.T
    b2 = fc2_b.reshape(1, 10)

    out = pl.pallas_call(
        _fused_kernel,
        out_shape=jax.ShapeDtypeStruct((Bp, 10), jnp.float32),
        grid=(Bp // BT,),
        in_specs=[
            pl.BlockSpec((2, 14, BT, 28), lambda i: (0, 0, i, 0)),
            pl.BlockSpec((150, NCOL), lambda i: (0, 0)),
            pl.BlockSpec((1, 260), lambda i: (0, 0)),
            pl.BlockSpec((13, 260, 100), lambda i: (0, 0, 0)),
            pl.BlockSpec((1, 100), lambda i: (0, 0)),
            pl.BlockSpec((100, 10), lambda i: (0, 0)),
            pl.BlockSpec((1, 10), lambda i: (0, 0)),
        ],
        out_specs=pl.BlockSpec((BT, 10), lambda i: (i, 0)),
        scratch_shapes=[pltpu.VMEM((15, BT, 30), jnp.float32),
                        pltpu.VMEM((15, BT, 30), jnp.float32)],
        compiler_params=pltpu.CompilerParams(
            dimension_semantics=("parallel",)),
    )(xd, ws, bc, w1r, b1, w2, b2)
    return out[:B]


def kernel(x, conv_w, conv_b, fc1_w, fc1_b, fc2_w, fc2_b):
    return _forward(x, conv_w, conv_b, fc1_w, fc1_b, fc2_w, fc2_b)


# speedup vs baseline: 1.1124x; 1.1124x over previous
import jax
import jax.numpy as jnp
from jax.experimental import pallas as pl
from jax.experimental.pallas import tpu as pltpu

BT = 256
PH = 384
NCOL = 2 * PH - 124


def _fused_kernel(xd_ref, ws_ref, bc_ref, w1_ref, b1_ref, w2_ref, b2_ref,
                  o_ref):
    bt = o_ref.shape[0]
    s_all = jnp.concatenate(
        [jnp.concatenate(
            [xd_ref[0, 0:13], xd_ref[1, 0:13], xd_ref[0, 1:14],
             xd_ref[1, 1:14], xd_ref[0, 2:15]], axis=2),
         jnp.concatenate(
            [xd_ref[1, 0:13], xd_ref[0, 1:14], xd_ref[1, 1:14],
             xd_ref[0, 2:15], xd_ref[1, 2:15]], axis=2)], axis=0)

    cc = jnp.dot(s_all.reshape(26 * bt, 150), ws_ref[...],
                 preferred_element_type=jnp.float32)

    t = jnp.maximum(cc[:13 * bt], cc[13 * bt:])
    m = jnp.maximum(t[:, 0:260], t[:, PH:PH + 260])
    m = jnp.maximum(m + bc_ref[...], 0.0)

    mi = m.reshape(13, bt, 260)
    h = b1_ref[...]
    for i in range(13):
        h = h + jnp.dot(mi[i], w1_ref[i],
                        preferred_element_type=jnp.float32)
    h = jnp.maximum(h, 0.0)
    o_ref[...] = (jnp.dot(h, w2_ref[...], preferred_element_type=jnp.float32)
                  + b2_ref[...])


@jax.jit
def _forward(x, conv_w, conv_b, fc1_w, fc1_b, fc2_w, fc2_b):
    B = x.shape[0]
    x3 = x.reshape(B, 28, 28)
    Bp = -(-B // BT) * BT
    if Bp != B:
        x3 = jnp.pad(x3, ((0, Bp - B), (0, 0), (0, 0)))
    xp = jnp.pad(x3, ((0, 0), (1, 1), (1, 1)))
    xd = xp.reshape(Bp, 15, 2, 30).transpose(2, 1, 0, 3)

    wc = conv_w.reshape(20, 5, 5)
    c = jnp.arange(20)[:, None, None, None, None]
    j = jnp.arange(13)[None, :, None, None, None]
    ph = jnp.arange(2)[None, None, :, None, None]
    ky = jnp.arange(5)[None, None, None, :, None]
    kx = jnp.arange(5)[None, None, None, None, :]
    full = (20, 13, 2, 5, 5)
    rows = jnp.broadcast_to(ky * 30 + 2 * j + ph + kx, full)
    cols = jnp.broadcast_to(ph * PH + c * 13 + j, full)
    vals = jnp.broadcast_to(wc[c, ky, kx], full)
    ws = jnp.zeros((150, NCOL), jnp.float32).at[
        rows.reshape(-1), cols.reshape(-1)].set(vals.reshape(-1))

    bc = jnp.broadcast_to(conv_b[:, None], (20, 13)).reshape(1, 260)

    w1r = fc1_w.reshape(100, 20, 13, 13).transpose(2, 1, 3, 0)
    w1r = w1r.reshape(13, 260, 100)
    b1 = fc1_b.reshape(1, 100)
    w2 = fc2_w.T
    b2 = fc2_b.reshape(1, 10)

    out = pl.pallas_call(
        _fused_kernel,
        out_shape=jax.ShapeDtypeStruct((Bp, 10), jnp.float32),
        grid=(Bp // BT,),
        in_specs=[
            pl.BlockSpec((2, 15, BT, 30), lambda i: (0, 0, i, 0)),
            pl.BlockSpec((150, NCOL), lambda i: (0, 0)),
            pl.BlockSpec((1, 260), lambda i: (0, 0)),
            pl.BlockSpec((13, 260, 100), lambda i: (0, 0, 0)),
            pl.BlockSpec((1, 100), lambda i: (0, 0)),
            pl.BlockSpec((100, 10), lambda i: (0, 0)),
            pl.BlockSpec((1, 10), lambda i: (0, 0)),
        ],
        out_specs=pl.BlockSpec((BT, 10), lambda i: (i, 0)),
        compiler_params=pltpu.CompilerParams(
            dimension_semantics=("parallel",)),
    )(xd, ws, bc, w1r, b1, w2, b2)
    return out[:B]


def kernel(x, conv_w, conv_b, fc1_w, fc1_b, fc2_w, fc2_b):
    return _forward(x, conv_w, conv_b, fc1_w, fc1_b, fc2_w, fc2_b)


# speedup vs baseline: 1.1341x; 1.0195x over previous
import jax
import jax.numpy as jnp
from jax.experimental import pallas as pl
from jax.experimental.pallas import tpu as pltpu

BT = 512
PH = 384
NCOL = 2 * PH - 124


def _fused_kernel(xd_ref, ws_ref, bc_ref, w1_ref, b1_ref, w2_ref, b2_ref,
                  o_ref):
    bt = o_ref.shape[0]
    s_all = jnp.concatenate(
        [jnp.concatenate(
            [xd_ref[0, 0:13], xd_ref[1, 0:13], xd_ref[0, 1:14],
             xd_ref[1, 1:14], xd_ref[0, 2:15]], axis=2),
         jnp.concatenate(
            [xd_ref[1, 0:13], xd_ref[0, 1:14], xd_ref[1, 1:14],
             xd_ref[0, 2:15], xd_ref[1, 2:15]], axis=2)], axis=0)

    cc = jnp.dot(s_all.reshape(26 * bt, 150), ws_ref[...],
                 preferred_element_type=jnp.float32)

    t = jnp.maximum(cc[:13 * bt], cc[13 * bt:])
    m = jnp.maximum(t[:, 0:260], t[:, PH:PH + 260])
    m = jnp.maximum(m + bc_ref[...], 0.0)

    mi = m.reshape(13, bt, 260)
    h = b1_ref[...]
    for i in range(13):
        h = h + jnp.dot(mi[i], w1_ref[i],
                        preferred_element_type=jnp.float32)
    h = jnp.maximum(h, 0.0)
    o_ref[...] = (jnp.dot(h, w2_ref[...], preferred_element_type=jnp.float32)
                  + b2_ref[...])


@jax.jit
def _forward(x, conv_w, conv_b, fc1_w, fc1_b, fc2_w, fc2_b):
    B = x.shape[0]
    x3 = x.reshape(B, 28, 28)
    Bp = -(-B // BT) * BT
    if Bp != B:
        x3 = jnp.pad(x3, ((0, Bp - B), (0, 0), (0, 0)))
    xp = jnp.pad(x3, ((0, 0), (1, 1), (1, 1)))
    xd = xp.reshape(Bp, 15, 2, 30).transpose(2, 1, 0, 3)

    wc = conv_w.reshape(20, 5, 5)
    c = jnp.arange(20)[:, None, None, None, None]
    j = jnp.arange(13)[None, :, None, None, None]
    ph = jnp.arange(2)[None, None, :, None, None]
    ky = jnp.arange(5)[None, None, None, :, None]
    kx = jnp.arange(5)[None, None, None, None, :]
    full = (20, 13, 2, 5, 5)
    rows = jnp.broadcast_to(ky * 30 + 2 * j + ph + kx, full)
    cols = jnp.broadcast_to(ph * PH + c * 13 + j, full)
    vals = jnp.broadcast_to(wc[c, ky, kx], full)
    ws = jnp.zeros((150, NCOL), jnp.float32).at[
        rows.reshape(-1), cols.reshape(-1)].set(vals.reshape(-1))

    bc = jnp.broadcast_to(conv_b[:, None], (20, 13)).reshape(1, 260)

    w1r = fc1_w.reshape(100, 20, 13, 13).transpose(2, 1, 3, 0)
    w1r = w1r.reshape(13, 260, 100)
    b1 = fc1_b.reshape(1, 100)
    w2 = fc2_w.T
    b2 = fc2_b.reshape(1, 10)

    out = pl.pallas_call(
        _fused_kernel,
        out_shape=jax.ShapeDtypeStruct((Bp, 10), jnp.float32),
        grid=(Bp // BT,),
        in_specs=[
            pl.BlockSpec((2, 15, BT, 30), lambda i: (0, 0, i, 0)),
            pl.BlockSpec((150, NCOL), lambda i: (0, 0)),
            pl.BlockSpec((1, 260), lambda i: (0, 0)),
            pl.BlockSpec((13, 260, 100), lambda i: (0, 0, 0)),
            pl.BlockSpec((1, 100), lambda i: (0, 0)),
            pl.BlockSpec((100, 10), lambda i: (0, 0)),
            pl.BlockSpec((1, 10), lambda i: (0, 0)),
        ],
        out_specs=pl.BlockSpec((BT, 10), lambda i: (i, 0)),
        compiler_params=pltpu.CompilerParams(
            dimension_semantics=("parallel",)),
    )(xd, ws, bc, w1r, b1, w2, b2)
    return out[:B]


def kernel(x, conv_w, conv_b, fc1_w, fc1_b, fc2_w, fc2_b):
    return _forward(x, conv_w, conv_b, fc1_w, fc1_b, fc2_w, fc2_b)
